# Initial kernel scaffold; baseline (speedup 1.0000x reference)
import jax
import jax.numpy as jnp
from jax.experimental import pallas as pl


def kernel(words, table):
    raise NotImplementedError("write your pallas kernel here")


# speedup vs baseline: 1.8084x; 1.8084x over previous
import functools

import jax
import jax.numpy as jnp
from jax import lax
from jax.experimental import pallas as pl
from jax.experimental.pallas import tpu as pltpu
from jax.experimental.pallas import tpu_sc as plsc

N_EMB = 32
SEQ = 50
NROWS = 16384
B_TOTAL = NROWS * SEQ

_info = plsc.get_sparse_core_info()
NUM_CORES = _info.num_cores
NUM_SUBCORES = _info.num_subcores
NW = NUM_CORES * NUM_SUBCORES
B_PER_W = B_TOTAL // NW
ROWS_PER_W = NROWS // NW
CHUNK_ROWS = 32
CHUNK = CHUNK_ROWS * SEQ
NCHUNK = ROWS_PER_W // CHUNK_ROWS
OUTER = NCHUNK // 2


@functools.partial(
    pl.kernel,
    mesh=plsc.VectorSubcoreMesh(core_axis_name="c", subcore_axis_name="s"),
    out_type=jax.ShapeDtypeStruct((NROWS, SEQ, N_EMB), jnp.float32),
    scratch_types=[
        pltpu.VMEM((B_PER_W,), jnp.int32),
        pltpu.VMEM((CHUNK, N_EMB), jnp.float32),
        pltpu.VMEM((CHUNK, N_EMB), jnp.float32),
        pltpu.SemaphoreType.DMA,
        pltpu.SemaphoreType.DMA,
        pltpu.SemaphoreType.DMA,
        pltpu.SemaphoreType.DMA,
    ],
    compiler_params=pltpu.CompilerParams(use_tc_tiling_on_sc=False),
)
def _gather_all(words_hbm, table_hbm, out_hbm, idx_v, rows0, rows1, g0, g1, w0, w1):
    wid = lax.axis_index("s") * NUM_CORES + lax.axis_index("c")
    base = wid * B_PER_W
    row_base = wid * ROWS_PER_W
    rows = (rows0, rows1)
    gsem = (g0, g1)
    wsem = (w0, w1)

    pltpu.sync_copy(words_hbm.at[pl.ds(base, B_PER_W)], idx_v)

    def gather_start(i, b):
        pltpu.async_copy(table_hbm.at[idx_v.at[pl.ds(i * CHUNK, CHUNK)]],
                         rows[b], gsem[b])

    def gather_wait(i, b):
        pltpu.make_async_copy(table_hbm.at[idx_v.at[pl.ds(i * CHUNK, CHUNK)]],
                              rows[b], gsem[b]).wait()

    def wb_start(i, b):
        for j in range(CHUNK_ROWS):
            pltpu.async_copy(rows[b].at[pl.ds(j * SEQ, SEQ)],
                             out_hbm.at[row_base + i * CHUNK_ROWS + j],
                             wsem[b])

    def wb_wait(i, b):
        for j in range(CHUNK_ROWS):
            pltpu.make_async_copy(rows[b].at[pl.ds(j * SEQ, SEQ)],
                                  out_hbm.at[row_base + i * CHUNK_ROWS + j],
                                  wsem[b]).wait()

    gather_start(0, 0)

    def body(t, carry):
        i0 = t * 2
        gather_wait(i0, 0)
        gather_start(i0 + 1, 1)
        wb_start(i0, 0)
        wb_wait(i0, 0)

        @pl.when(t + 1 < OUTER)
        def _():
            gather_start(i0 + 2, 0)

        gather_wait(i0 + 1, 1)
        wb_start(i0 + 1, 1)
        wb_wait(i0 + 1, 1)
        return carry

    lax.fori_loop(0, OUTER, body, 0)


def kernel(words, table):
    flat = words.reshape(B_TOTAL).astype(jnp.int32)
    return _gather_all(flat, table)
